# baseline (speedup 1.0000x reference)
import functools

import jax
import jax.numpy as jnp
from jax import lax
from jax.experimental import pallas as pl
from jax.experimental.pallas import tpu as pltpu
from jax.experimental.pallas import tpu_sc as plsc

N_TOK = 2048
D_MODEL = 1024
D_FF = 2048
N_EXP = 8
HID = 128
TOPK = 2

N_PAIR = N_TOK * TOPK
BG = 768
G_MAX = (N_PAIR + N_EXP * (BG - 1) + BG - 1) // BG
N_SLOT = G_MAX * BG

SC_WORKERS = 32
PAIRS_PER_W = N_PAIR // SC_WORKERS
SC_CHUNK = 64
SC_STEPS = PAIRS_PER_W // SC_CHUNK

D_HALF = D_MODEL // 2


def _pack_bf16(yb):
    lo = jax.lax.bitcast_convert_type(yb[:, :D_HALF], jnp.uint16).astype(jnp.int32)
    hi = jax.lax.bitcast_convert_type(yb[:, D_HALF:], jnp.uint16).astype(jnp.int32)
    return lo | (hi << 16)


def _unpack_bf16(v):
    lo = jax.lax.bitcast_convert_type((v & 0xFFFF).astype(jnp.uint16),
                                      jnp.bfloat16)
    hi = jax.lax.bitcast_convert_type(
        jax.lax.shift_right_logical(v, 16).astype(jnp.uint16), jnp.bfloat16)
    return jnp.concatenate([lo, hi], axis=1)


def _router_kernel(x_ref, rW1_ref, rb1_ref, rW2_ref, rb2_ref,
                   pos_ref, w_ref, be_ref, nact_ref, xp_ref):
    x = x_ref[...]
    xp_ref[...] = _pack_bf16(x.astype(jnp.bfloat16))
    h = jnp.dot(x, rW1_ref[...], preferred_element_type=jnp.float32)
    h = jnp.maximum(h + rb1_ref[...], 0.0)
    logits = jnp.dot(h, rW2_ref[...], preferred_element_type=jnp.float32)
    logits = logits + rb2_ref[...]

    col8 = jax.lax.broadcasted_iota(jnp.int32, (N_TOK, N_EXP), 1)
    l0 = jnp.max(logits, axis=1, keepdims=True)
    e0 = jnp.min(jnp.where(logits == l0, col8, N_EXP), axis=1, keepdims=True)
    masked = jnp.where(col8 == e0, -jnp.inf, logits)
    l1 = jnp.max(masked, axis=1, keepdims=True)
    e1 = jnp.min(jnp.where(masked == l1, col8, N_EXP), axis=1, keepdims=True)
    w0 = 1.0 / (1.0 + jnp.exp(l1 - l0))
    w1 = 1.0 - w0
    w_ref[...] = jnp.concatenate([w0, w1], axis=1)

    oh0 = (col8 == e0).astype(jnp.float32)
    oh1 = (col8 == e1).astype(jnp.float32)
    oh01 = jnp.concatenate([oh0, oh1], axis=1)
    v = oh01
    d = 1
    while d < N_TOK:
        shifted = jnp.concatenate(
            [jnp.zeros((d, 2 * N_EXP), jnp.float32), v[:N_TOK - d]], axis=0)
        v = v + shifted
        d *= 2
    csum = v - oh01

    cnt0 = jnp.sum(oh0, axis=0, keepdims=True)
    cnt1 = jnp.sum(oh1, axis=0, keepdims=True)
    counts = cnt0 + cnt1
    padded = jnp.floor((counts + (BG - 1)) * (1.0 / BG)) * BG

    v = padded
    for d in (1, 2, 4):
        shifted = jnp.concatenate(
            [jnp.zeros((1, d), jnp.float32), v[:, :N_EXP - d]], axis=1)
        v = v + shifted
    start = v - padded

    rank0 = jnp.sum(csum[:, :N_EXP] * oh0, axis=1, keepdims=True)
    rank1 = jnp.sum(csum[:, N_EXP:] * oh1, axis=1, keepdims=True)
    cnt0_at_e1 = jnp.sum(cnt0 * oh1, axis=1, keepdims=True)
    start0 = jnp.sum(start * oh0, axis=1, keepdims=True)
    start1 = jnp.sum(start * oh1, axis=1, keepdims=True)
    pos0 = start0 + rank0
    pos1 = start1 + cnt0_at_e1 + rank1
    pos_ref[...] = jnp.concatenate([pos0, pos1], axis=1).astype(jnp.int32)

    gslot = jax.lax.broadcasted_iota(jnp.int32, (G_MAX, N_EXP), 0).astype(jnp.float32) * BG
    e_iota = jax.lax.broadcasted_iota(jnp.int32, (G_MAX, N_EXP), 1).astype(jnp.float32)
    startb = jnp.broadcast_to(start, (G_MAX, N_EXP))
    paddedb = jnp.broadcast_to(padded, (G_MAX, N_EXP))
    inb = jnp.logical_and(gslot >= startb, gslot < startb + paddedb)
    be_ref[...] = jnp.sum(jnp.where(inb, e_iota, 0.0), axis=1,
                          keepdims=True).astype(jnp.int32)
    nact_ref[...] = (jnp.sum(padded, axis=1, keepdims=True) *
                     (1.0 / BG)).astype(jnp.int32)


def _router_call(x, rW1, rb1, rW2, rb2):
    return pl.pallas_call(
        _router_kernel,
        out_shape=(
            jax.ShapeDtypeStruct((N_TOK, TOPK), jnp.int32),
            jax.ShapeDtypeStruct((N_TOK, TOPK), jnp.float32),
            jax.ShapeDtypeStruct((G_MAX, 1), jnp.int32),
            jax.ShapeDtypeStruct((1, 1), jnp.int32),
            jax.ShapeDtypeStruct((N_TOK, D_HALF), jnp.int32),
        ),
    )(x, rW1, rb1.reshape(1, HID), rW2, rb2.reshape(1, N_EXP))


@functools.lru_cache(maxsize=None)
def _sc_kernels():
    mesh = plsc.VectorSubcoreMesh(core_axis_name="c", subcore_axis_name="s")
    sc_scratch = [
        pltpu.VMEM((SC_STEPS, SC_CHUNK), jnp.int32),
        pltpu.VMEM((SC_CHUNK, D_HALF), jnp.int32),
        pltpu.VMEM((SC_CHUNK, D_HALF), jnp.int32),
        pltpu.SemaphoreType.DMA,
        pltpu.SemaphoreType.DMA,
        pltpu.SemaphoreType.DMA,
        pltpu.SemaphoreType.DMA,
    ]

    @functools.partial(
        pl.kernel,
        out_type=jax.ShapeDtypeStruct((N_SLOT, D_HALF), jnp.int32),
        mesh=mesh,
        scratch_types=sc_scratch,
    )
    def _dispatch_sc(xp_hbm, pos_hbm, xg_hbm, idx_v, r0, r1, si0, si1, so0, so1):
        wid = lax.axis_index("s") * 2 + lax.axis_index("c")
        pltpu.sync_copy(pos_hbm.at[wid], idx_v)
        bufs, isem, osem = (r0, r1), (si0, si1), (so0, so1)

        def src(ci):
            t = lax.rem(wid * PAIRS_PER_W + ci * SC_CHUNK, N_TOK)
            return xp_hbm.at[pl.ds(t, SC_CHUNK)]

        cin = [pltpu.make_async_copy(src(ci), bufs[ci % 2], isem[ci % 2])
               for ci in range(SC_STEPS)]
        cout = [pltpu.make_async_copy(bufs[ci % 2], xg_hbm.at[idx_v.at[ci]],
                                      osem[ci % 2])
                for ci in range(SC_STEPS)]
        for c in cin:
            c.start()
        for ci in range(SC_STEPS):
            cin[ci].wait()
            cout[ci].start()
        for ci in range(SC_STEPS):
            cout[ci].wait()

    @functools.partial(
        pl.kernel,
        out_type=jax.ShapeDtypeStruct((N_PAIR, D_HALF), jnp.int32),
        mesh=mesh,
        scratch_types=sc_scratch,
    )
    def _gather_sc(yg_hbm, pos_hbm, gboth_hbm, idx_v, r0, r1, si0, si1, so0, so1):
        wid = lax.axis_index("s") * 2 + lax.axis_index("c")
        pltpu.sync_copy(pos_hbm.at[wid], idx_v)
        bufs, isem, osem = (r0, r1), (si0, si1), (so0, so1)

        def dst(ci):
            return gboth_hbm.at[pl.ds(wid * PAIRS_PER_W + ci * SC_CHUNK,
                                      SC_CHUNK)]

        cin = [pltpu.make_async_copy(yg_hbm.at[idx_v.at[ci]], bufs[ci % 2],
                                     isem[ci % 2])
               for ci in range(SC_STEPS)]
        cout = [pltpu.make_async_copy(bufs[ci % 2], dst(ci), osem[ci % 2])
                for ci in range(SC_STEPS)]
        for c in cin:
            c.start()
        for ci in range(SC_STEPS):
            cin[ci].wait()
            cout[ci].start()
        for ci in range(SC_STEPS):
            cout[ci].wait()

    return _dispatch_sc, _gather_sc


def _gffn_kernel(be_ref, nact_ref, xg_ref, w1_ref, b1_ref, w2_ref, b2_ref,
                 yg_ref, w1q_ref, w2q_ref):
    g = pl.program_id(0)
    active = g < nact_ref[0]
    new_expert = jnp.logical_or(g == 0,
                                be_ref[g] != be_ref[jnp.maximum(g - 1, 0)])

    @pl.when(jnp.logical_and(active, new_expert))
    def _():
        w1q_ref[...] = w1_ref[0].astype(jnp.bfloat16)
        w2q_ref[...] = w2_ref[0].astype(jnp.bfloat16)

    @pl.when(active)
    def _():
        xq = _unpack_bf16(xg_ref[...])
        h = jnp.dot(xq, w1q_ref[...], preferred_element_type=jnp.float32)
        h = jnp.maximum(h + b1_ref[0], 0.0).astype(jnp.bfloat16)
        y = jnp.dot(h, w2q_ref[...], preferred_element_type=jnp.float32)
        yg_ref[...] = _pack_bf16((y + b2_ref[0]).astype(jnp.bfloat16))


def _gffn_call(be, nact, xg, eW1, eb1, eW2, eb2):
    grid_spec = pltpu.PrefetchScalarGridSpec(
        num_scalar_prefetch=2,
        grid=(G_MAX,),
        in_specs=[
            pl.BlockSpec((BG, D_HALF), lambda g, be, na: (g, 0)),
            pl.BlockSpec((1, D_MODEL, D_FF), lambda g, be, na: (be[g], 0, 0)),
            pl.BlockSpec((1, 1, D_FF), lambda g, be, na: (be[g], 0, 0)),
            pl.BlockSpec((1, D_FF, D_MODEL), lambda g, be, na: (be[g], 0, 0)),
            pl.BlockSpec((1, 1, D_MODEL), lambda g, be, na: (be[g], 0, 0)),
        ],
        out_specs=pl.BlockSpec((BG, D_HALF), lambda g, be, na: (g, 0)),
        scratch_shapes=[
            pltpu.VMEM((D_MODEL, D_FF), jnp.bfloat16),
            pltpu.VMEM((D_FF, D_MODEL), jnp.bfloat16),
        ],
    )
    return pl.pallas_call(
        _gffn_kernel,
        grid_spec=grid_spec,
        out_shape=jax.ShapeDtypeStruct((N_SLOT, D_HALF), jnp.int32),
    )(be, nact, xg, eW1, eb1.reshape(N_EXP, 1, D_FF), eW2,
      eb2.reshape(N_EXP, 1, D_MODEL))


def _combine_kernel(g_ref, w_ref, out_ref):
    g0 = _unpack_bf16(g_ref[0:N_TOK]).astype(jnp.float32)
    g1 = _unpack_bf16(g_ref[N_TOK:N_PAIR]).astype(jnp.float32)
    out_ref[...] = g0 * w_ref[:, 0:1] + g1 * w_ref[:, 1:2]


def _combine_call(gboth, w):
    return pl.pallas_call(
        _combine_kernel,
        out_shape=jax.ShapeDtypeStruct((N_TOK, D_MODEL), jnp.float32),
    )(gboth, w)


def kernel(x, rW1, rb1, rW2, rb2, eW1, eb1, eW2, eb2):
    dispatch_sc, gather_sc = _sc_kernels()
    pos, w, be, nact, xp = _router_call(x, rW1, rb1, rW2, rb2)
    pos3d = pos.T.reshape(SC_WORKERS, SC_STEPS, SC_CHUNK)
    xg = dispatch_sc(xp, pos3d)
    yg = _gffn_call(be.reshape(G_MAX), nact.reshape(1), xg, eW1, eb1, eW2, eb2)
    gboth = gather_sc(yg, pos3d)
    return _combine_call(gboth, w)


# speedup vs baseline: 1.0869x; 1.0869x over previous
import functools

import jax
import jax.numpy as jnp
from jax import lax
from jax.experimental import pallas as pl
from jax.experimental.pallas import tpu as pltpu
from jax.experimental.pallas import tpu_sc as plsc

N_TOK = 2048
D_MODEL = 1024
D_FF = 2048
N_EXP = 8
HID = 128
TOPK = 2

N_PAIR = N_TOK * TOPK
BG = 512
G_MAX = (N_PAIR + N_EXP * (BG - 1) + BG - 1) // BG
N_SLOT = G_MAX * BG

SC_WORKERS = 32
PAIRS_PER_W = N_PAIR // SC_WORKERS
SC_CHUNK = 64
SC_STEPS = PAIRS_PER_W // SC_CHUNK

D_HALF = D_MODEL // 2


def _pack_bf16(yb):
    lo = jax.lax.bitcast_convert_type(yb[:, :D_HALF], jnp.uint16).astype(jnp.int32)
    hi = jax.lax.bitcast_convert_type(yb[:, D_HALF:], jnp.uint16).astype(jnp.int32)
    return lo | (hi << 16)


def _unpack_bf16(v):
    lo = jax.lax.bitcast_convert_type((v & 0xFFFF).astype(jnp.uint16),
                                      jnp.bfloat16)
    hi = jax.lax.bitcast_convert_type(
        jax.lax.shift_right_logical(v, 16).astype(jnp.uint16), jnp.bfloat16)
    return jnp.concatenate([lo, hi], axis=1)


def _router_kernel(x_ref, rW1_ref, rb1_ref, rW2_ref, rb2_ref,
                   pos_ref, w_ref, be_ref, nact_ref, xp_ref):
    x = x_ref[...]
    xp_ref[...] = _pack_bf16(x.astype(jnp.bfloat16))
    h = jnp.dot(x, rW1_ref[...], preferred_element_type=jnp.float32)
    h = jnp.maximum(h + rb1_ref[...], 0.0)
    logits = jnp.dot(h, rW2_ref[...], preferred_element_type=jnp.float32)
    logits = logits + rb2_ref[...]

    col8 = jax.lax.broadcasted_iota(jnp.int32, (N_TOK, N_EXP), 1)
    l0 = jnp.max(logits, axis=1, keepdims=True)
    e0 = jnp.min(jnp.where(logits == l0, col8, N_EXP), axis=1, keepdims=True)
    masked = jnp.where(col8 == e0, -jnp.inf, logits)
    l1 = jnp.max(masked, axis=1, keepdims=True)
    e1 = jnp.min(jnp.where(masked == l1, col8, N_EXP), axis=1, keepdims=True)
    w0 = 1.0 / (1.0 + jnp.exp(l1 - l0))
    w1 = 1.0 - w0
    w_ref[...] = jnp.concatenate([w0, w1], axis=1)

    oh0 = (col8 == e0).astype(jnp.float32)
    oh1 = (col8 == e1).astype(jnp.float32)
    oh01 = jnp.concatenate([oh0, oh1], axis=1)
    v = oh01
    d = 1
    while d < N_TOK:
        shifted = jnp.concatenate(
            [jnp.zeros((d, 2 * N_EXP), jnp.float32), v[:N_TOK - d]], axis=0)
        v = v + shifted
        d *= 2
    csum = v - oh01

    cnt0 = jnp.sum(oh0, axis=0, keepdims=True)
    cnt1 = jnp.sum(oh1, axis=0, keepdims=True)
    counts = cnt0 + cnt1
    padded = jnp.floor((counts + (BG - 1)) * (1.0 / BG)) * BG

    v = padded
    for d in (1, 2, 4):
        shifted = jnp.concatenate(
            [jnp.zeros((1, d), jnp.float32), v[:, :N_EXP - d]], axis=1)
        v = v + shifted
    start = v - padded

    rank0 = jnp.sum(csum[:, :N_EXP] * oh0, axis=1, keepdims=True)
    rank1 = jnp.sum(csum[:, N_EXP:] * oh1, axis=1, keepdims=True)
    cnt0_at_e1 = jnp.sum(cnt0 * oh1, axis=1, keepdims=True)
    start0 = jnp.sum(start * oh0, axis=1, keepdims=True)
    start1 = jnp.sum(start * oh1, axis=1, keepdims=True)
    pos0 = start0 + rank0
    pos1 = start1 + cnt0_at_e1 + rank1
    pos_ref[...] = jnp.concatenate([pos0, pos1], axis=1).astype(jnp.int32)

    gslot = jax.lax.broadcasted_iota(jnp.int32, (G_MAX, N_EXP), 0).astype(jnp.float32) * BG
    e_iota = jax.lax.broadcasted_iota(jnp.int32, (G_MAX, N_EXP), 1).astype(jnp.float32)
    startb = jnp.broadcast_to(start, (G_MAX, N_EXP))
    paddedb = jnp.broadcast_to(padded, (G_MAX, N_EXP))
    inb = jnp.logical_and(gslot >= startb, gslot < startb + paddedb)
    be_ref[...] = jnp.sum(jnp.where(inb, e_iota, 0.0), axis=1,
                          keepdims=True).astype(jnp.int32)
    nact_ref[...] = (jnp.sum(padded, axis=1, keepdims=True) *
                     (1.0 / BG)).astype(jnp.int32)


def _router_call(x, rW1, rb1, rW2, rb2):
    return pl.pallas_call(
        _router_kernel,
        out_shape=(
            jax.ShapeDtypeStruct((N_TOK, TOPK), jnp.int32),
            jax.ShapeDtypeStruct((N_TOK, TOPK), jnp.float32),
            jax.ShapeDtypeStruct((G_MAX, 1), jnp.int32),
            jax.ShapeDtypeStruct((1, 1), jnp.int32),
            jax.ShapeDtypeStruct((N_TOK, D_HALF), jnp.int32),
        ),
    )(x, rW1, rb1.reshape(1, HID), rW2, rb2.reshape(1, N_EXP))


@functools.lru_cache(maxsize=None)
def _sc_kernels():
    mesh = plsc.VectorSubcoreMesh(core_axis_name="c", subcore_axis_name="s")
    sc_scratch = [
        pltpu.VMEM((SC_STEPS, SC_CHUNK), jnp.int32),
        pltpu.VMEM((SC_CHUNK, D_HALF), jnp.int32),
        pltpu.VMEM((SC_CHUNK, D_HALF), jnp.int32),
        pltpu.SemaphoreType.DMA,
        pltpu.SemaphoreType.DMA,
        pltpu.SemaphoreType.DMA,
        pltpu.SemaphoreType.DMA,
    ]

    @functools.partial(
        pl.kernel,
        out_type=jax.ShapeDtypeStruct((N_SLOT, D_HALF), jnp.int32),
        mesh=mesh,
        scratch_types=sc_scratch,
    )
    def _dispatch_sc(xp_hbm, pos_hbm, xg_hbm, idx_v, r0, r1, si0, si1, so0, so1):
        wid = lax.axis_index("s") * 2 + lax.axis_index("c")
        pltpu.sync_copy(pos_hbm.at[wid], idx_v)
        bufs, isem, osem = (r0, r1), (si0, si1), (so0, so1)

        def src(ci):
            t = lax.rem(wid * PAIRS_PER_W + ci * SC_CHUNK, N_TOK)
            return xp_hbm.at[pl.ds(t, SC_CHUNK)]

        cin = [pltpu.make_async_copy(src(ci), bufs[ci % 2], isem[ci % 2])
               for ci in range(SC_STEPS)]
        cout = [pltpu.make_async_copy(bufs[ci % 2], xg_hbm.at[idx_v.at[ci]],
                                      osem[ci % 2])
                for ci in range(SC_STEPS)]
        for c in cin:
            c.start()
        for ci in range(SC_STEPS):
            cin[ci].wait()
            cout[ci].start()
        for ci in range(SC_STEPS):
            cout[ci].wait()

    @functools.partial(
        pl.kernel,
        out_type=jax.ShapeDtypeStruct((N_PAIR, D_HALF), jnp.int32),
        mesh=mesh,
        scratch_types=sc_scratch,
    )
    def _gather_sc(yg_hbm, pos_hbm, gboth_hbm, idx_v, r0, r1, si0, si1, so0, so1):
        wid = lax.axis_index("s") * 2 + lax.axis_index("c")
        pltpu.sync_copy(pos_hbm.at[wid], idx_v)
        bufs, isem, osem = (r0, r1), (si0, si1), (so0, so1)

        def dst(ci):
            return gboth_hbm.at[pl.ds(wid * PAIRS_PER_W + ci * SC_CHUNK,
                                      SC_CHUNK)]

        cin = [pltpu.make_async_copy(yg_hbm.at[idx_v.at[ci]], bufs[ci % 2],
                                     isem[ci % 2])
               for ci in range(SC_STEPS)]
        cout = [pltpu.make_async_copy(bufs[ci % 2], dst(ci), osem[ci % 2])
                for ci in range(SC_STEPS)]
        for c in cin:
            c.start()
        for ci in range(SC_STEPS):
            cin[ci].wait()
            cout[ci].start()
        for ci in range(SC_STEPS):
            cout[ci].wait()

    return _dispatch_sc, _gather_sc


def _gffn_kernel(be_ref, nact_ref, xg_ref, w1_ref, b1_ref, w2_ref, b2_ref,
                 yg_ref, w1q_ref, w2q_ref):
    g = pl.program_id(0)
    active = g < nact_ref[0]
    new_expert = jnp.logical_or(g == 0,
                                be_ref[g] != be_ref[jnp.maximum(g - 1, 0)])

    @pl.when(jnp.logical_and(active, new_expert))
    def _():
        w1q_ref[...] = w1_ref[0].astype(jnp.bfloat16)
        w2q_ref[...] = w2_ref[0].astype(jnp.bfloat16)

    @pl.when(active)
    def _():
        xq = _unpack_bf16(xg_ref[...])
        h = jnp.dot(xq, w1q_ref[...], preferred_element_type=jnp.float32)
        h = jnp.maximum(h + b1_ref[0], 0.0).astype(jnp.bfloat16)
        y = jnp.dot(h, w2q_ref[...], preferred_element_type=jnp.float32)
        yg_ref[...] = _pack_bf16((y + b2_ref[0]).astype(jnp.bfloat16))


def _gffn_call(be, nact, xg, eW1, eb1, eW2, eb2):
    grid_spec = pltpu.PrefetchScalarGridSpec(
        num_scalar_prefetch=2,
        grid=(G_MAX,),
        in_specs=[
            pl.BlockSpec((BG, D_HALF), lambda g, be, na: (g, 0)),
            pl.BlockSpec((1, D_MODEL, D_FF), lambda g, be, na: (be[g], 0, 0)),
            pl.BlockSpec((1, 1, D_FF), lambda g, be, na: (be[g], 0, 0)),
            pl.BlockSpec((1, D_FF, D_MODEL), lambda g, be, na: (be[g], 0, 0)),
            pl.BlockSpec((1, 1, D_MODEL), lambda g, be, na: (be[g], 0, 0)),
        ],
        out_specs=pl.BlockSpec((BG, D_HALF), lambda g, be, na: (g, 0)),
        scratch_shapes=[
            pltpu.VMEM((D_MODEL, D_FF), jnp.bfloat16),
            pltpu.VMEM((D_FF, D_MODEL), jnp.bfloat16),
        ],
    )
    return pl.pallas_call(
        _gffn_kernel,
        grid_spec=grid_spec,
        out_shape=jax.ShapeDtypeStruct((N_SLOT, D_HALF), jnp.int32),
    )(be, nact, xg, eW1, eb1.reshape(N_EXP, 1, D_FF), eW2,
      eb2.reshape(N_EXP, 1, D_MODEL))


def _combine_kernel(g_ref, w_ref, out_ref):
    g0 = _unpack_bf16(g_ref[0:N_TOK]).astype(jnp.float32)
    g1 = _unpack_bf16(g_ref[N_TOK:N_PAIR]).astype(jnp.float32)
    out_ref[...] = g0 * w_ref[:, 0:1] + g1 * w_ref[:, 1:2]


def _combine_call(gboth, w):
    return pl.pallas_call(
        _combine_kernel,
        out_shape=jax.ShapeDtypeStruct((N_TOK, D_MODEL), jnp.float32),
    )(gboth, w)


def kernel(x, rW1, rb1, rW2, rb2, eW1, eb1, eW2, eb2):
    dispatch_sc, gather_sc = _sc_kernels()
    pos, w, be, nact, xp = _router_call(x, rW1, rb1, rW2, rb2)
    pos3d = pos.T.reshape(SC_WORKERS, SC_STEPS, SC_CHUNK)
    xg = dispatch_sc(xp, pos3d)
    yg = _gffn_call(be.reshape(G_MAX), nact.reshape(1), xg, eW1, eb1, eW2, eb2)
    gboth = gather_sc(yg, pos3d)
    return _combine_call(gboth, w)
